# baseline (speedup 1.0000x reference)
import jax
import jax.numpy as jnp
from jax.experimental import pallas as pl

_B, _N, _G = 8, 4096, 128
_C = 8
_FH, _FW = 8, 8
_ZW, _ZH = 1.1, 1.1
_BLK = 2048
_NBLK = _N // _BLK


def _body(pk_ref, gt_ref, lab_ref, off_ref, mask_ref):
    pk = pk_ref[0]
    gtt = jnp.transpose(gt_ref[0])

    idxf = pk[4:5]
    flgf = pk[5:6]
    iota_g = jax.lax.broadcasted_iota(jnp.int32, (_G, _BLK), 0).astype(jnp.float32)
    oh = (iota_g == idxf)
    ga = jnp.dot(gtt, oh.astype(jnp.float32),
                 preferred_element_type=jnp.float32,
                 precision=jax.lax.Precision.HIGHEST)

    x1, y1, x2, y2 = pk[0:1], pk[1:2], pk[2:3], pk[3:4]
    cx = (x1 + x2) / 2.0
    cy = (y1 + y2) / 2.0
    w = (x2 - x1) * _ZW
    h = (y2 - y1) * _ZH
    ax1 = cx - w / 2.0
    ay1 = cy - h / 2.0
    ax2 = cx + w / 2.0
    ay2 = cy + h / 2.0

    gx1, gy1, gx2, gy2, lbl = ga[0:1], ga[1:2], ga[2:3], ga[3:4], ga[4:5]
    rx1 = gx1 - ax1
    ry1 = gy1 - ay1
    rx2 = gx2 - ax1
    ry2 = gy2 - ay1
    rw = rx2 - rx1
    rh = ry2 - ry1
    rcx = (rx1 + rx2) / 2.0
    rcy = (ry1 + ry2) / 2.0
    sw = (ax2 - ax1) / _FW
    sh = (ay2 - ay1) / _FH
    w_sigma = rw / 2.0 / sw
    h_sigma = rh / 2.0 / sh
    pw = rcx / sw
    ph = rcy / sh
    q1 = rx1 / sw
    q2 = ry1 / sh
    q3 = rx2 / sw
    q4 = ry2 / sh

    def e3(v):
        return v[:, None, :]

    ind_w = jax.lax.broadcasted_iota(jnp.int32, (_FH, _FW, _BLK), 1).astype(jnp.float32)
    ind_h = jax.lax.broadcasted_iota(jnp.int32, (_FH, _FW, _BLK), 0).astype(jnp.float32)
    w_term = jnp.square((e3(pw) - ind_w - 0.5) / e3(w_sigma))
    h_term = jnp.square((e3(ph) - ind_h - 0.5) / e3(h_sigma))
    g = jnp.exp(-(w_term + h_term))
    cond = ((jnp.abs(ind_w + 0.5 - e3(pw)) < e3(w_sigma))
            & (jnp.abs(ind_h + 0.5 - e3(ph)) < e3(h_sigma)))
    g = jnp.where(cond, g, 0.0)
    lab_ref[...] = jnp.broadcast_to(g[None], (_C, _FH, _FW, _BLK))

    ox1 = e3(q1) - (ind_w + 0.5)
    oy1 = e3(q2) - (ind_h + 0.5)
    ox2 = e3(q3) - (ind_w + 0.5)
    oy2 = e3(q4) - (ind_h + 0.5)
    off_ref[...] = jnp.stack([ox1, oy1, ox2, oy2], axis=0)

    cls = jax.lax.broadcasted_iota(jnp.int32, (_C, _BLK), 0).astype(jnp.float32)
    pos = flgf > 0.0
    nn = jnp.where(flgf != 0.0, lbl, 0.0)
    mone = pos & (nn > 0.0)
    m = (cls == (jnp.abs(lbl) - 1.0)) & mone
    mask_ref[...] = m


def kernel(boxes, gt_boxes, match_pos_flag, match_gt_id):
    pk = jnp.concatenate(
        [boxes.transpose(0, 2, 1),
         match_gt_id.astype(jnp.float32)[:, None, :],
         match_pos_flag.astype(jnp.float32)[:, None, :]], axis=1)
    labT, offT, maskT = pl.pallas_call(
        _body,
        grid=(_B, _NBLK),
        in_specs=[
            pl.BlockSpec((1, 6, _BLK), lambda b, j: (b, 0, j)),
            pl.BlockSpec((1, _G, 5), lambda b, j: (b, 0, 0)),
        ],
        out_specs=[
            pl.BlockSpec((_C, _FH, _FW, _BLK),
                         lambda b, j: (0, 0, 0, b * _NBLK + j)),
            pl.BlockSpec((4, _FH, _FW, _BLK),
                         lambda b, j: (0, 0, 0, b * _NBLK + j)),
            pl.BlockSpec((_C, _BLK), lambda b, j: (0, b * _NBLK + j)),
        ],
        out_shape=[
            jax.ShapeDtypeStruct((_C, _FH, _FW, _B * _N), jnp.float32),
            jax.ShapeDtypeStruct((4, _FH, _FW, _B * _N), jnp.float32),
            jax.ShapeDtypeStruct((_C, _B * _N), jnp.bool_),
        ],
    )(pk, gt_boxes)
    lab = labT.transpose(3, 0, 1, 2)
    off = offT.transpose(3, 0, 1, 2)
    mask = maskT.transpose(1, 0)
    return lab, off, mask


# speedup vs baseline: 1.0003x; 1.0003x over previous
import jax
import jax.numpy as jnp
from jax.experimental import pallas as pl

_B, _N, _G = 8, 4096, 128
_C = 8
_FH, _FW = 8, 8
_ZW, _ZH = 1.1, 1.1
_BLK = 2048
_NBLK = _N // _BLK


def _body(pk_ref, gt_ref, lab_ref, off_ref, mask_ref):
    pk = pk_ref[0]
    gtt = jnp.transpose(gt_ref[0])

    idxf = pk[4:5]
    flgf = pk[5:6]
    iota_g = jax.lax.broadcasted_iota(jnp.int32, (_G, _BLK), 0).astype(jnp.float32)
    oh = (iota_g == idxf)
    ga = jnp.dot(gtt, oh.astype(jnp.float32),
                 preferred_element_type=jnp.float32,
                 precision=jax.lax.Precision.HIGHEST)

    x1, y1, x2, y2 = pk[0:1], pk[1:2], pk[2:3], pk[3:4]
    cx = (x1 + x2) / 2.0
    cy = (y1 + y2) / 2.0
    w = (x2 - x1) * _ZW
    h = (y2 - y1) * _ZH
    ax1 = cx - w / 2.0
    ay1 = cy - h / 2.0
    ax2 = cx + w / 2.0
    ay2 = cy + h / 2.0

    gx1, gy1, gx2, gy2, lbl = ga[0:1], ga[1:2], ga[2:3], ga[3:4], ga[4:5]
    rx1 = gx1 - ax1
    ry1 = gy1 - ay1
    rx2 = gx2 - ax1
    ry2 = gy2 - ay1
    rw = rx2 - rx1
    rh = ry2 - ry1
    rcx = (rx1 + rx2) / 2.0
    rcy = (ry1 + ry2) / 2.0
    sw = (ax2 - ax1) / _FW
    sh = (ay2 - ay1) / _FH
    w_sigma = rw / 2.0 / sw
    h_sigma = rh / 2.0 / sh
    pw = rcx / sw
    ph = rcy / sh
    q1 = rx1 / sw
    q2 = ry1 / sh
    q3 = rx2 / sw
    q4 = ry2 / sh

    def e3(v):
        return v[:, None, :]

    ind_w = jax.lax.broadcasted_iota(jnp.int32, (_FH, _FW, _BLK), 1).astype(jnp.float32)
    ind_h = jax.lax.broadcasted_iota(jnp.int32, (_FH, _FW, _BLK), 0).astype(jnp.float32)
    w_term = jnp.square((e3(pw) - ind_w - 0.5) / e3(w_sigma))
    h_term = jnp.square((e3(ph) - ind_h - 0.5) / e3(h_sigma))
    g = jnp.exp(-(w_term + h_term))
    cond = ((jnp.abs(ind_w + 0.5 - e3(pw)) < e3(w_sigma))
            & (jnp.abs(ind_h + 0.5 - e3(ph)) < e3(h_sigma)))
    g = jnp.where(cond, g, 0.0)
    lab_ref[...] = jnp.broadcast_to(g[None], (_C, _FH, _FW, _BLK))

    ox1 = e3(q1) - (ind_w + 0.5)
    oy1 = e3(q2) - (ind_h + 0.5)
    ox2 = e3(q3) - (ind_w + 0.5)
    oy2 = e3(q4) - (ind_h + 0.5)
    off_ref[...] = jnp.stack([ox1, oy1, ox2, oy2], axis=0)

    cls = jax.lax.broadcasted_iota(jnp.int32, (_C, _BLK), 0).astype(jnp.float32)
    pos = flgf > 0.0
    nn = jnp.where(flgf != 0.0, lbl, 0.0)
    mone = pos & (nn > 0.0)
    m = (cls == (jnp.abs(lbl) - 1.0)) & mone
    mask_ref[...] = m.astype(jnp.int8)


def kernel(boxes, gt_boxes, match_pos_flag, match_gt_id):
    pk = jnp.concatenate(
        [boxes.transpose(0, 2, 1),
         match_gt_id.astype(jnp.float32)[:, None, :],
         match_pos_flag.astype(jnp.float32)[:, None, :]], axis=1)
    labT, offT, maskT = pl.pallas_call(
        _body,
        grid=(_B, _NBLK),
        in_specs=[
            pl.BlockSpec((1, 6, _BLK), lambda b, j: (b, 0, j)),
            pl.BlockSpec((1, _G, 5), lambda b, j: (b, 0, 0)),
        ],
        out_specs=[
            pl.BlockSpec((_C, _FH, _FW, _BLK),
                         lambda b, j: (0, 0, 0, b * _NBLK + j)),
            pl.BlockSpec((4, _FH, _FW, _BLK),
                         lambda b, j: (0, 0, 0, b * _NBLK + j)),
            pl.BlockSpec((_C, _BLK), lambda b, j: (0, b * _NBLK + j)),
        ],
        out_shape=[
            jax.ShapeDtypeStruct((_C, _FH, _FW, _B * _N), jnp.float32),
            jax.ShapeDtypeStruct((4, _FH, _FW, _B * _N), jnp.float32),
            jax.ShapeDtypeStruct((_C, _B * _N), jnp.int8),
        ],
    )(pk, gt_boxes)
    lab = labT.transpose(3, 0, 1, 2)
    off = offT.transpose(3, 0, 1, 2)
    mask = maskT.transpose(1, 0).astype(bool)
    return lab, off, mask


# speedup vs baseline: 1.0297x; 1.0294x over previous
import jax
import jax.numpy as jnp
from jax.experimental import pallas as pl

_B, _N, _G = 8, 4096, 128
_C = 8
_FH, _FW = 8, 8
_ZW, _ZH = 1.1, 1.1
_BLK = 2048
_NBLK = _N // _BLK


def _body(pk_ref, gt_ref, lab_ref, off_ref, mask_ref):
    pk = pk_ref[0]
    gtt = gt_ref[0]

    idxf = pk[4:5]
    flgf = pk[5:6]
    iota_g = jax.lax.broadcasted_iota(jnp.int32, (_G, _BLK), 0).astype(jnp.float32)
    oh = (iota_g == idxf)
    ga = jnp.dot(gtt, oh.astype(jnp.float32),
                 preferred_element_type=jnp.float32,
                 precision=jax.lax.Precision.HIGHEST)

    x1, y1, x2, y2 = pk[0:1], pk[1:2], pk[2:3], pk[3:4]
    cx = (x1 + x2) / 2.0
    cy = (y1 + y2) / 2.0
    w = (x2 - x1) * _ZW
    h = (y2 - y1) * _ZH
    ax1 = cx - w / 2.0
    ay1 = cy - h / 2.0
    ax2 = cx + w / 2.0
    ay2 = cy + h / 2.0

    gx1, gy1, gx2, gy2, lbl = ga[0:1], ga[1:2], ga[2:3], ga[3:4], ga[4:5]
    rx1 = gx1 - ax1
    ry1 = gy1 - ay1
    rx2 = gx2 - ax1
    ry2 = gy2 - ay1
    rw = rx2 - rx1
    rh = ry2 - ry1
    rcx = (rx1 + rx2) / 2.0
    rcy = (ry1 + ry2) / 2.0
    sw = (ax2 - ax1) / _FW
    sh = (ay2 - ay1) / _FH
    w_sigma = rw / 2.0 / sw
    h_sigma = rh / 2.0 / sh
    pw = rcx / sw
    ph = rcy / sh
    q1 = rx1 / sw
    q2 = ry1 / sh
    q3 = rx2 / sw
    q4 = ry2 / sh

    def e3(v):
        return v[:, None, :]

    ind_w = jax.lax.broadcasted_iota(jnp.int32, (_FH, _FW, _BLK), 1).astype(jnp.float32)
    ind_h = jax.lax.broadcasted_iota(jnp.int32, (_FH, _FW, _BLK), 0).astype(jnp.float32)
    w_term = jnp.square((e3(pw) - ind_w - 0.5) / e3(w_sigma))
    h_term = jnp.square((e3(ph) - ind_h - 0.5) / e3(h_sigma))
    g = jnp.exp(-(w_term + h_term))
    cond = ((jnp.abs(ind_w + 0.5 - e3(pw)) < e3(w_sigma))
            & (jnp.abs(ind_h + 0.5 - e3(ph)) < e3(h_sigma)))
    g = jnp.where(cond, g, 0.0)
    lab_ref[...] = jnp.broadcast_to(g[None], (_C, _FH, _FW, _BLK))

    ox1 = e3(q1) - (ind_w + 0.5)
    oy1 = e3(q2) - (ind_h + 0.5)
    ox2 = e3(q3) - (ind_w + 0.5)
    oy2 = e3(q4) - (ind_h + 0.5)
    off_ref[...] = jnp.stack([ox1, oy1, ox2, oy2], axis=0)

    cls = jax.lax.broadcasted_iota(jnp.int32, (_C, _BLK), 0).astype(jnp.float32)
    pos = flgf > 0.0
    nn = jnp.where(flgf != 0.0, lbl, 0.0)
    mone = pos & (nn > 0.0)
    m = (cls == (jnp.abs(lbl) - 1.0)) & mone
    mask_ref[...] = m.astype(jnp.int8)


def kernel(boxes, gt_boxes, match_pos_flag, match_gt_id):
    pk = jnp.concatenate(
        [boxes.transpose(0, 2, 1),
         match_gt_id.astype(jnp.float32)[:, None, :],
         match_pos_flag.astype(jnp.float32)[:, None, :]], axis=1)
    gtt = gt_boxes.transpose(0, 2, 1)
    labT, offT, maskT = pl.pallas_call(
        _body,
        grid=(_B, _NBLK),
        in_specs=[
            pl.BlockSpec((1, 6, _BLK), lambda b, j: (b, 0, j)),
            pl.BlockSpec((1, 5, _G), lambda b, j: (b, 0, 0)),
        ],
        out_specs=[
            pl.BlockSpec((_C, _FH, _FW, _BLK),
                         lambda b, j: (0, 0, 0, b * _NBLK + j)),
            pl.BlockSpec((4, _FH, _FW, _BLK),
                         lambda b, j: (0, 0, 0, b * _NBLK + j)),
            pl.BlockSpec((_C, _BLK), lambda b, j: (0, b * _NBLK + j)),
        ],
        out_shape=[
            jax.ShapeDtypeStruct((_C, _FH, _FW, _B * _N), jnp.float32),
            jax.ShapeDtypeStruct((4, _FH, _FW, _B * _N), jnp.float32),
            jax.ShapeDtypeStruct((_C, _B * _N), jnp.int8),
        ],
    )(pk, gtt)
    lab = labT.transpose(3, 0, 1, 2)
    off = offT.transpose(3, 0, 1, 2)
    mask = maskT.transpose(1, 0).astype(bool)
    return lab, off, mask
